# baseline (speedup 1.0000x reference)
import functools

import jax
import jax.numpy as jnp
from jax import lax
from jax.experimental import pallas as pl
from jax.experimental.pallas import tpu as pltpu
from jax.experimental.pallas import tpu_sc as plsc

_N_FRAMES = 1000
_C, _H, _W = 1, 192, 256
_B = 64

_B_SC = 16
_B_TC = _B - _B_SC
_G = 8

_INFO = plsc.get_sparse_core_info()
_NC = _INFO.num_cores
_NS = _INFO.num_subcores
_L = _INFO.num_lanes
_NW = _NC * _NS

_mesh = plsc.VectorSubcoreMesh(core_axis_name="c", subcore_axis_name="s")


@functools.partial(
    pl.kernel,
    mesh=_mesh,
    out_type=jax.ShapeDtypeStruct((_B_SC, _H, _W), jnp.float32),
    scratch_types=[
        pltpu.VMEM((1, 1), jnp.int32),
        pltpu.VMEM((1, _H, _W), jnp.float32),
        pltpu.SemaphoreType.DMA,
    ],
)
def _sc_gather_elu(idx_hbm, table_hbm, out_hbm, idx_v, row_v, sem):
    wid = lax.axis_index("s") * _NC + lax.axis_index("c")

    @pl.when(wid < _B_SC)
    def _():
        pltpu.sync_copy(idx_hbm.at[wid], idx_v)
        pltpu.async_copy(table_hbm.at[idx_v.at[0]], row_v, sem).wait()

        @plsc.parallel_loop(0, _H, unroll=2)
        def _(h):
            for j in range(_W // _L):
                x = row_v[0, h, pl.ds(j * _L, _L)]
                row_v[0, h, pl.ds(j * _L, _L)] = jnp.maximum(x, 0.0) + jnp.exp(
                    jnp.minimum(x, 0.0)
                )

        pltpu.sync_copy(row_v, out_hbm.at[pl.ds(wid, 1)])


def _tc_body(idx_ref, *refs):
    in_refs = refs[:_G]
    out_ref = refs[_G]
    for k in range(_G):
        x = in_refs[k][...]
        out_ref[k, :, :] = jnp.maximum(x[0], 0.0) + jnp.exp(jnp.minimum(x[0], 0.0))


def _mk_spec(k):
    return pl.BlockSpec((1, _H, _W), lambda i, idx, k=k: (idx[i * _G + k], 0, 0))


_tc_gather_elu = pl.pallas_call(
    _tc_body,
    grid_spec=pltpu.PrefetchScalarGridSpec(
        num_scalar_prefetch=1,
        grid=(_B_TC // _G,),
        in_specs=[_mk_spec(k) for k in range(_G)],
        out_specs=pl.BlockSpec((_G, _H, _W), lambda i, idx: (i, 0, 0)),
    ),
    out_shape=jax.ShapeDtypeStruct((_B_TC, _H, _W), jnp.float32),
)


def kernel(indices, maps):
    idx = indices.astype(jnp.int32)
    table = maps.reshape(_N_FRAMES, _H, _W)
    idx_sc = idx[_B_TC:].reshape(_B_SC, 1, 1)
    out_sc = _sc_gather_elu(idx_sc, table)
    out_tc = _tc_gather_elu(idx[:_B_TC], *([table] * _G))
    out = jnp.concatenate([out_tc, out_sc], axis=0)
    return out.reshape(_B, _C, _H, _W)


# speedup vs baseline: 3.6484x; 3.6484x over previous
import functools

import jax
import jax.numpy as jnp
from jax.experimental import pallas as pl
from jax.experimental.pallas import tpu as pltpu

_N_FRAMES = 1000
_C, _H, _W = 1, 192, 256
_B = 64
_G = 16


def _tc_body(idx_ref, *refs):
    in_refs = refs[:_G]
    out_ref = refs[_G]
    for k in range(_G):
        x = in_refs[k][...]
        out_ref[k, :, :] = jnp.maximum(x[0], 0.0) + jnp.exp(jnp.minimum(x[0], 0.0))


def _mk_spec(k):
    return pl.BlockSpec((1, _H, _W), lambda i, idx, k=k: (idx[i * _G + k], 0, 0))


def kernel(indices, maps):
    idx = indices.astype(jnp.int32)
    table = maps.reshape(_N_FRAMES, _H, _W)
    out = pl.pallas_call(
        _tc_body,
        grid_spec=pltpu.PrefetchScalarGridSpec(
            num_scalar_prefetch=1,
            grid=(_B // _G,),
            in_specs=[_mk_spec(k) for k in range(_G)],
            out_specs=pl.BlockSpec((_G, _H, _W), lambda i, idx: (i, 0, 0)),
        ),
        out_shape=jax.ShapeDtypeStruct((_B, _H, _W), jnp.float32),
    )(idx, *([table] * _G))
    return out.reshape(_B, _C, _H, _W)


# speedup vs baseline: 4.1516x; 1.1379x over previous
import functools

import jax
import jax.numpy as jnp
from jax.experimental import pallas as pl
from jax.experimental.pallas import tpu as pltpu

_N_FRAMES = 1000
_C, _H, _W = 1, 192, 256
_B = 64
_G = 32


def _tc_body(idx_ref, *refs):
    in_refs = refs[:_G]
    out_ref = refs[_G]
    for k in range(_G):
        x = in_refs[k][...]
        out_ref[k, :, :] = jnp.maximum(x[0], 0.0) + jnp.exp(jnp.minimum(x[0], 0.0))


def _mk_spec(k):
    return pl.BlockSpec((1, _H, _W), lambda i, idx, k=k: (idx[i * _G + k], 0, 0))


def kernel(indices, maps):
    idx = indices.astype(jnp.int32)
    table = maps.reshape(_N_FRAMES, _H, _W)
    out = pl.pallas_call(
        _tc_body,
        grid_spec=pltpu.PrefetchScalarGridSpec(
            num_scalar_prefetch=1,
            grid=(_B // _G,),
            in_specs=[_mk_spec(k) for k in range(_G)],
            out_specs=pl.BlockSpec((_G, _H, _W), lambda i, idx: (i, 0, 0)),
        ),
        out_shape=jax.ShapeDtypeStruct((_B, _H, _W), jnp.float32),
    )(idx, *([table] * _G))
    return out.reshape(_B, _C, _H, _W)
